# Initial kernel scaffold; baseline (speedup 1.0000x reference)
import jax
import jax.numpy as jnp
from jax.experimental import pallas as pl


def kernel(node_energy, node_attrs, ptr, edge_index, batch, node_level, scale, shift):
    raise NotImplementedError("write your pallas kernel here")


# speedup vs baseline: 68.5946x; 68.5946x over previous
import functools

import jax
import jax.numpy as jnp
from jax import lax
from jax.experimental import pallas as pl
from jax.experimental.pallas import tpu as pltpu
from jax.experimental.pallas import tpu_sc as plsc

NC = 2
NS = 16
NW = NC * NS
LN = 16

_MESH = plsc.VectorSubcoreMesh(
    core_axis_name="c", subcore_axis_name="s", num_cores=NC, num_subcores=NS
)
_PARAMS = pltpu.CompilerParams(needs_layout_passes=False)


def _wid():
    return lax.axis_index("s") * NC + lax.axis_index("c")


def _make_edge_hist(n_nodes: int, n_edges: int, n_graphs: int):
    epw = n_edges // NW
    chunk = 10000
    assert epw % chunk == 0 and chunk % LN == 0 and epw % 8 == 0

    @functools.partial(
        pl.kernel,
        out_type=jax.ShapeDtypeStruct((NW, n_graphs), jnp.int32),
        mesh=_MESH,
        compiler_params=_PARAMS,
        scratch_types=[
            pltpu.VMEM((n_nodes,), jnp.int32),
            pltpu.VMEM((chunk,), jnp.int32),
            pltpu.VMEM((LN * n_graphs,), jnp.int32),
            pltpu.VMEM((n_graphs,), jnp.int32),
            pltpu.SemaphoreType.DMA,
        ],
    )
    def edge_hist(batch_hbm, dst_hbm, out_hbm, batch_v, edges_v, hist_v, row_v, sem):
        wid = _wid()
        zeros = jnp.zeros((LN,), jnp.int32)
        ones = jnp.ones((LN,), jnp.int32)
        lane = lax.iota(jnp.int32, LN)
        lane_g = lane * n_graphs

        def zbody(i, _):
            hist_v[pl.ds(i * LN, LN)] = zeros
            return 0

        lax.fori_loop(0, (LN * n_graphs) // LN, zbody, 0)

        pltpu.sync_copy(batch_hbm, batch_v)

        base = wid * epw

        def chunk_body(c, _):
            pltpu.async_copy(
                dst_hbm.at[pl.ds(base + c * chunk, chunk)], edges_v, sem
            ).wait()

            def inner(i, _):
                idx = edges_v[pl.ds(i * LN, LN)]
                vals = plsc.load_gather(batch_v, [idx])
                plsc.addupdate_scatter(hist_v, [lane_g + vals], ones)
                return 0

            lax.fori_loop(0, chunk // LN, inner, 0)
            return 0

        lax.fori_loop(0, epw // chunk, chunk_body, 0)

        def red_body(g, _):
            def lbody(l, acc):
                return acc + hist_v[pl.ds(l * n_graphs + g * LN, LN)]

            row_v[pl.ds(g * LN, LN)] = lax.fori_loop(0, LN, lbody, zeros)
            return 0

        lax.fori_loop(0, n_graphs // LN, red_body, 0)
        pltpu.sync_copy(row_v, out_hbm.at[wid])

    return edge_hist


def _make_node_phase(n_nodes: int, n_graphs: int, ptr_pad: int):
    npt = (n_nodes // NW) // LN * LN
    tail = n_nodes - NW * npt
    assert npt % 8 == 0 and tail % LN == 0
    nbuf = npt + tail

    @functools.partial(
        pl.kernel,
        out_type=jax.ShapeDtypeStruct((n_nodes,), jnp.float32),
        mesh=_MESH,
        compiler_params=_PARAMS,
        scratch_types=[
            pltpu.VMEM((NW * n_graphs,), jnp.int32),
            pltpu.VMEM((n_graphs,), jnp.int32),
            pltpu.VMEM((ptr_pad,), jnp.int32),
            pltpu.VMEM((64,), jnp.float32),
            pltpu.VMEM((nbuf,), jnp.int32),
            pltpu.VMEM((nbuf,), jnp.int32),
            pltpu.VMEM((nbuf,), jnp.float32),
            pltpu.VMEM((nbuf * 10,), jnp.float32),
            pltpu.VMEM((nbuf,), jnp.float32),
            pltpu.SemaphoreType.DMA,
        ],
    )
    def node_phase(
        part_hbm, ptr_hbm, coef_hbm, batch_hbm, level_hbm, energy_hbm,
        attrs_hbm, out_hbm, part_v, mask_v, ptr_v, coef_v, batch_v, level_v,
        energy_v, attrs_v, out_v, sem,
    ):
        wid = _wid()
        zeros = jnp.zeros((LN,), jnp.int32)
        fzeros = jnp.zeros((LN,), jnp.float32)
        lane = lax.iota(jnp.int32, LN)
        lane10 = lane * 10

        pltpu.sync_copy(part_hbm, part_v)
        pltpu.sync_copy(ptr_hbm, ptr_v)
        pltpu.sync_copy(coef_hbm, coef_v)

        def mask_body(g, _):
            def rbody(r, acc):
                return acc + part_v[pl.ds(r * n_graphs + g * LN, LN)]

            ne = lax.fori_loop(0, NW, rbody, zeros)
            nn = ptr_v[pl.ds(g * LN + 1, LN)] - ptr_v[pl.ds(g * LN, LN)]
            iso = ((nn == 1) & (ne == 0)).astype(jnp.int32)
            mask_v[pl.ds(g * LN, LN)] = iso
            return 0

        lax.fori_loop(0, n_graphs // LN, mask_body, 0)

        nbase = wid * npt
        pltpu.sync_copy(batch_hbm.at[pl.ds(nbase, npt)], batch_v.at[pl.ds(0, npt)])
        pltpu.sync_copy(level_hbm.at[pl.ds(nbase, npt)], level_v.at[pl.ds(0, npt)])
        pltpu.sync_copy(energy_hbm.at[pl.ds(nbase, npt)], energy_v.at[pl.ds(0, npt)])
        pltpu.sync_copy(
            attrs_hbm.at[pl.ds(nbase * 10, npt * 10)], attrs_v.at[pl.ds(0, npt * 10)]
        )

        tbase = NW * npt

        @pl.when(wid == NW - 1)
        def _():
            pltpu.sync_copy(
                batch_hbm.at[pl.ds(tbase, tail)], batch_v.at[pl.ds(npt, tail)]
            )
            pltpu.sync_copy(
                level_hbm.at[pl.ds(tbase, tail)], level_v.at[pl.ds(npt, tail)]
            )
            pltpu.sync_copy(
                energy_hbm.at[pl.ds(tbase, tail)], energy_v.at[pl.ds(npt, tail)]
            )
            pltpu.sync_copy(
                attrs_hbm.at[pl.ds(tbase * 10, tail * 10)],
                attrs_v.at[pl.ds(npt * 10, tail * 10)],
            )

        def node_body(j, _):
            sl = pl.ds(j * LN, LN)
            iso = plsc.load_gather(mask_v, [batch_v[sl]])
            lvl16 = level_v[sl] * 16
            accs = fzeros
            acch = fzeros
            for z in range(10):
                a = plsc.load_gather(attrs_v, [lane10 + (j * (LN * 10) + z)])
                cs = plsc.load_gather(coef_v, [lvl16 + z])
                ch = plsc.load_gather(coef_v, [lvl16 + (32 + z)])
                accs = accs + a * cs
                acch = acch + a * ch
            res = energy_v[sl] * accs + acch
            out_v[sl] = jnp.where(iso == 1, fzeros, res)
            return 0

        lax.fori_loop(0, npt // LN, node_body, 0)

        @pl.when(wid == NW - 1)
        def _():
            lax.fori_loop(npt // LN, nbuf // LN, node_body, 0)

        pltpu.sync_copy(out_v.at[pl.ds(0, npt)], out_hbm.at[pl.ds(nbase, npt)])

        @pl.when(wid == NW - 1)
        def _():
            pltpu.sync_copy(
                out_v.at[pl.ds(npt, tail)], out_hbm.at[pl.ds(tbase, tail)]
            )

    return node_phase


def kernel(node_energy, node_attrs, ptr, edge_index, batch, node_level, scale, shift):
    n_nodes = node_energy.shape[0]
    n_edges = edge_index.shape[1]
    n_graphs = ptr.shape[0] - 1

    dst = edge_index[1]
    ptr_pad = (ptr.shape[0] + 15) // 16 * 16
    ptr_p = jnp.pad(ptr, (0, ptr_pad - ptr.shape[0]))
    z = scale.shape[1]
    sc_p = jnp.pad(scale, ((0, 0), (0, 16 - z))).reshape(-1)
    sh_p = jnp.pad(shift, ((0, 0), (0, 16 - z))).reshape(-1)
    coef = jnp.concatenate([sc_p, sh_p])
    attrs_flat = node_attrs.reshape(-1)

    partials = _make_edge_hist(n_nodes, n_edges, n_graphs)(batch, dst)
    out = _make_node_phase(n_nodes, n_graphs, ptr_pad)(
        partials.reshape(-1), ptr_p, coef, batch, node_level, node_energy,
        attrs_flat,
    )
    return out


# speedup vs baseline: 88.4445x; 1.2894x over previous
import functools

import jax
import jax.numpy as jnp
from jax import lax
from jax.experimental import pallas as pl
from jax.experimental.pallas import tpu as pltpu
from jax.experimental.pallas import tpu_sc as plsc

NC = 2
NS = 16
NW = NC * NS
LN = 16

_MESH = plsc.VectorSubcoreMesh(
    core_axis_name="c", subcore_axis_name="s", num_cores=NC, num_subcores=NS
)
_PARAMS = pltpu.CompilerParams(needs_layout_passes=False)


def _wid():
    return lax.axis_index("s") * NC + lax.axis_index("c")


def _make_edge_hist(n_nodes: int, n_edges: int, n_graphs: int):
    epw = n_edges // NW
    chunk = 10000
    unroll = 5
    assert epw % chunk == 0 and chunk % (LN * unroll) == 0 and epw % 8 == 0

    @functools.partial(
        pl.kernel,
        out_type=jax.ShapeDtypeStruct((NW, n_graphs), jnp.int32),
        mesh=_MESH,
        compiler_params=_PARAMS,
        scratch_types=[
            pltpu.VMEM((n_nodes,), jnp.int32),
            pltpu.VMEM((chunk,), jnp.int32),
            pltpu.VMEM((LN * n_graphs,), jnp.int32),
            pltpu.VMEM((n_graphs,), jnp.int32),
            pltpu.SemaphoreType.DMA,
        ],
    )
    def edge_hist(batch_hbm, ei_hbm, out_hbm, batch_v, edges_v, hist_v, row_v, sem):
        wid = _wid()
        zeros = jnp.zeros((LN,), jnp.int32)
        ones = jnp.ones((LN,), jnp.int32)
        lane = lax.iota(jnp.int32, LN)
        lane_g = lane * n_graphs

        def zbody(i, _):
            for u in range(8):
                hist_v[pl.ds(i * (8 * LN) + u * LN, LN)] = zeros
            return 0

        lax.fori_loop(0, (LN * n_graphs) // (8 * LN), zbody, 0)

        pltpu.sync_copy(batch_hbm, batch_v)

        base = wid * epw

        def chunk_body(c, _):
            pltpu.async_copy(
                ei_hbm.at[pl.ds(n_edges + base + c * chunk, chunk)], edges_v, sem
            ).wait()

            def inner(i, _):
                for u in range(unroll):
                    idx = edges_v[pl.ds(i * (LN * unroll) + u * LN, LN)]
                    vals = plsc.load_gather(batch_v, [idx])
                    plsc.addupdate_scatter(hist_v, [lane_g + vals], ones)
                return 0

            lax.fori_loop(0, chunk // (LN * unroll), inner, 0)
            return 0

        lax.fori_loop(0, epw // chunk, chunk_body, 0)

        def red_body(g, _):
            acc = zeros
            for l in range(LN):
                acc = acc + hist_v[pl.ds(l * n_graphs + g * LN, LN)]
            row_v[pl.ds(g * LN, LN)] = acc
            return 0

        lax.fori_loop(0, n_graphs // LN, red_body, 0)
        pltpu.sync_copy(row_v, out_hbm.at[wid])

    return edge_hist


def _make_node_phase(n_nodes: int, n_graphs: int, ptr_pad: int):
    npt = (n_nodes // NW) // LN * LN
    tail = n_nodes - NW * npt
    assert npt % 8 == 0 and tail % LN == 0
    nbuf = npt + tail

    @functools.partial(
        pl.kernel,
        out_type=jax.ShapeDtypeStruct((n_nodes,), jnp.float32),
        mesh=_MESH,
        compiler_params=_PARAMS,
        scratch_types=[
            pltpu.VMEM((NW * n_graphs,), jnp.int32),
            pltpu.VMEM((n_graphs,), jnp.int32),
            pltpu.VMEM((ptr_pad,), jnp.int32),
            pltpu.VMEM((64,), jnp.float32),
            pltpu.VMEM((nbuf,), jnp.int32),
            pltpu.VMEM((nbuf,), jnp.int32),
            pltpu.VMEM((nbuf,), jnp.float32),
            pltpu.VMEM((nbuf * 10,), jnp.float32),
            pltpu.VMEM((nbuf,), jnp.float32),
            pltpu.SemaphoreType.DMA,
        ],
    )
    def node_phase(
        part_hbm, ptr_hbm, coef_hbm, batch_hbm, level_hbm, energy_hbm,
        attrs_hbm, out_hbm, part_v, mask_v, ptr_v, coef_v, batch_v, level_v,
        energy_v, attrs_v, out_v, sem,
    ):
        wid = _wid()
        zeros = jnp.zeros((LN,), jnp.int32)
        fzeros = jnp.zeros((LN,), jnp.float32)
        lane = lax.iota(jnp.int32, LN)
        lane10 = lane * 10

        pltpu.sync_copy(part_hbm, part_v)
        pltpu.sync_copy(ptr_hbm, ptr_v)
        pltpu.sync_copy(coef_hbm, coef_v)

        def mask_body(g, _):
            ne = zeros
            for r in range(NW):
                ne = ne + part_v[pl.ds(r * n_graphs + g * LN, LN)]
            nn = ptr_v[pl.ds(g * LN + 1, LN)] - ptr_v[pl.ds(g * LN, LN)]
            iso = ((nn == 1) & (ne == 0)).astype(jnp.int32)
            mask_v[pl.ds(g * LN, LN)] = iso
            return 0

        lax.fori_loop(0, n_graphs // LN, mask_body, 0)

        nbase = wid * npt
        pltpu.sync_copy(batch_hbm.at[pl.ds(nbase, npt)], batch_v.at[pl.ds(0, npt)])
        pltpu.sync_copy(level_hbm.at[pl.ds(nbase, npt)], level_v.at[pl.ds(0, npt)])
        pltpu.sync_copy(energy_hbm.at[pl.ds(nbase, npt)], energy_v.at[pl.ds(0, npt)])
        pltpu.sync_copy(
            attrs_hbm.at[pl.ds(nbase * 10, npt * 10)], attrs_v.at[pl.ds(0, npt * 10)]
        )

        tbase = NW * npt

        @pl.when(wid == NW - 1)
        def _():
            pltpu.sync_copy(
                batch_hbm.at[pl.ds(tbase, tail)], batch_v.at[pl.ds(npt, tail)]
            )
            pltpu.sync_copy(
                level_hbm.at[pl.ds(tbase, tail)], level_v.at[pl.ds(npt, tail)]
            )
            pltpu.sync_copy(
                energy_hbm.at[pl.ds(tbase, tail)], energy_v.at[pl.ds(npt, tail)]
            )
            pltpu.sync_copy(
                attrs_hbm.at[pl.ds(tbase * 10, tail * 10)],
                attrs_v.at[pl.ds(npt * 10, tail * 10)],
            )

        def node_body(j, _):
            sl = pl.ds(j * LN, LN)
            iso = plsc.load_gather(mask_v, [batch_v[sl]])
            lvl16 = level_v[sl] * 16
            accs = fzeros
            acch = fzeros
            for z in range(10):
                a = plsc.load_gather(attrs_v, [lane10 + (j * (LN * 10) + z)])
                cs = plsc.load_gather(coef_v, [lvl16 + z])
                ch = plsc.load_gather(coef_v, [lvl16 + (32 + z)])
                accs = accs + a * cs
                acch = acch + a * ch
            res = energy_v[sl] * accs + acch
            out_v[sl] = jnp.where(iso == 1, fzeros, res)
            return 0

        lax.fori_loop(0, npt // LN, node_body, 0)

        @pl.when(wid == NW - 1)
        def _():
            lax.fori_loop(npt // LN, nbuf // LN, node_body, 0)

        pltpu.sync_copy(out_v.at[pl.ds(0, npt)], out_hbm.at[pl.ds(nbase, npt)])

        @pl.when(wid == NW - 1)
        def _():
            pltpu.sync_copy(
                out_v.at[pl.ds(npt, tail)], out_hbm.at[pl.ds(tbase, tail)]
            )

    return node_phase


def kernel(node_energy, node_attrs, ptr, edge_index, batch, node_level, scale, shift):
    n_nodes = node_energy.shape[0]
    n_edges = edge_index.shape[1]
    n_graphs = ptr.shape[0] - 1

    ptr_pad = (ptr.shape[0] + 15) // 16 * 16
    ptr_p = jnp.pad(ptr, (0, ptr_pad - ptr.shape[0]))
    z = scale.shape[1]
    sc_p = jnp.pad(scale, ((0, 0), (0, 16 - z))).reshape(-1)
    sh_p = jnp.pad(shift, ((0, 0), (0, 16 - z))).reshape(-1)
    coef = jnp.concatenate([sc_p, sh_p])
    attrs_flat = node_attrs.reshape(-1)

    partials = _make_edge_hist(n_nodes, n_edges, n_graphs)(batch, edge_index.reshape(-1))
    out = _make_node_phase(n_nodes, n_graphs, ptr_pad)(
        partials.reshape(-1), ptr_p, coef, batch, node_level, node_energy,
        attrs_flat,
    )
    return out
